# baseline (speedup 1.0000x reference)
import functools

import jax
import jax.numpy as jnp
from jax import lax
from jax.experimental import pallas as pl
from jax.experimental.pallas import tpu as pltpu
from jax.experimental.pallas import tpu_sc as plsc

_B = 1024
_H = 64
_NC = 2
_NS = 16
_NW = _NC * _NS
_BPW = _B // _NW
_LANES = 16

_BN = 512
_NBUF = 8

_sc_mesh = plsc.VectorSubcoreMesh(core_axis_name="c", subcore_axis_name="s")


@functools.partial(
    pl.kernel,
    mesh=_sc_mesh,
    out_type=jax.ShapeDtypeStruct((_B, 2 * _H), jnp.float32),
    scratch_types=[
        pltpu.VMEM((_BPW,), jnp.int32),
        pltpu.VMEM((_BPW,), jnp.int32),
        pltpu.VMEM((_BPW, 2 * _H), jnp.float32),
        pltpu.SemaphoreType.DMA,
    ],
)
def _sc_gather(table_hbm, idx_hbm, out_hbm, idx_v, pidx_v, rows_v, sem):
    wid = lax.axis_index("s") * _NC + lax.axis_index("c")
    base = wid * _BPW
    pltpu.sync_copy(idx_hbm.at[pl.ds(base, _BPW)], idx_v)

    @pl.loop(0, _BPW, step=_LANES)
    def _(i):
        slc = pl.ds(i, _LANES)
        pidx_v.at[slc][...] = lax.shift_right_logical(idx_v.at[slc][...], 1)

    pltpu.async_copy(table_hbm.at[pidx_v], rows_v, sem).wait()
    pltpu.sync_copy(rows_v, out_hbm.at[pl.ds(base, _BPW)])


def _select_half(g, xi):
    odd = (xi & 1) == 1
    return jnp.where(odd, g[:, _H:], g[:, :_H])


def _make_mm_body(ng):
    def _mm_body(g_ref, xi_ref, w_ref, o_hbm, h_ref, obuf, sems):
        i = pl.program_id(0)
        slot = lax.rem(i, _NBUF)

        @pl.when(i == 0)
        def _():
            h = _select_half(g_ref[...], xi_ref[...])
            h_ref[...] = h.astype(jnp.bfloat16)

        @pl.when(i >= _NBUF)
        def _():
            pltpu.make_async_copy(
                obuf.at[slot],
                o_hbm.at[:, pl.ds((i - _NBUF) * _BN, _BN)],
                sems.at[slot],
            ).wait()

        obuf[slot] = lax.dot_general(
            h_ref[...],
            w_ref[...].astype(jnp.bfloat16),
            dimension_numbers=(((1,), (1,)), ((), ())),
            preferred_element_type=jnp.float32,
        )
        pltpu.make_async_copy(
            obuf.at[slot],
            o_hbm.at[:, pl.ds(i * _BN, _BN)],
            sems.at[slot],
        ).start()

        @pl.when(i == ng - 1)
        def _():
            for k in range(_NBUF):
                j = ng - _NBUF + k
                pltpu.make_async_copy(
                    obuf.at[j % _NBUF],
                    o_hbm.at[:, pl.ds(j * _BN, _BN)],
                    sems.at[j % _NBUF],
                ).wait()

    return _mm_body


def _tail_body(prev_ref, g_ref, xi_ref, w_ref, o_ref):
    del prev_ref
    h = _select_half(g_ref[...], xi_ref[...])
    o_ref[...] = lax.dot_general(
        h.astype(jnp.bfloat16),
        w_ref[...].astype(jnp.bfloat16),
        dimension_numbers=(((1,), (1,)), ((), ())),
        preferred_element_type=jnp.float32,
    )


def kernel(x, emb, W):
    xi = x.astype(jnp.int32)
    xi2 = xi.reshape(_B, 1)
    table = emb.reshape(emb.shape[0] // 2, 2 * _H)
    g = _sc_gather(table, xi)
    V = W.shape[0]
    ng = (V // _BN)
    v_tail = V - ng * _BN

    main = pl.pallas_call(
        _make_mm_body(ng),
        grid=(ng,),
        in_specs=[
            pl.BlockSpec((_B, 2 * _H), lambda i: (0, 0)),
            pl.BlockSpec((_B, 1), lambda i: (0, 0)),
            pl.BlockSpec((_BN, _H), lambda i: (i, 0)),
        ],
        out_specs=pl.BlockSpec(memory_space=pl.ANY),
        out_shape=jax.ShapeDtypeStruct((_B, V), jnp.float32),
        scratch_shapes=[
            pltpu.VMEM((_B, _H), jnp.bfloat16),
            pltpu.VMEM((_NBUF, _B, _BN), jnp.float32),
            pltpu.SemaphoreType.DMA((_NBUF,)),
        ],
        compiler_params=pltpu.CompilerParams(
            dimension_semantics=("arbitrary",),
        ),
    )(g, xi2, W)

    bt = 256
    tail_block = ng * _BN // bt
    logits = pl.pallas_call(
        _tail_body,
        grid=(1,),
        in_specs=[
            pl.BlockSpec(memory_space=pl.ANY),
            pl.BlockSpec((_B, 2 * _H), lambda i: (0, 0)),
            pl.BlockSpec((_B, 1), lambda i: (0, 0)),
            pl.BlockSpec((bt, _H), lambda i: (tail_block, 0)),
        ],
        out_specs=pl.BlockSpec((_B, bt), lambda i: (0, tail_block)),
        out_shape=jax.ShapeDtypeStruct((_B, V), jnp.float32),
        input_output_aliases={0: 0},
    )(main, g, xi2, W)
    return logits


# speedup vs baseline: 1.0004x; 1.0004x over previous
import functools

import jax
import jax.numpy as jnp
from jax import lax
from jax.experimental import pallas as pl
from jax.experimental.pallas import tpu as pltpu
from jax.experimental.pallas import tpu_sc as plsc

_B = 1024
_H = 64
_NC = 2
_NS = 16
_NW = _NC * _NS
_BPW = _B // _NW
_LANES = 16

_BN = 512
_NBUF = 8

_sc_mesh = plsc.VectorSubcoreMesh(core_axis_name="c", subcore_axis_name="s")


@functools.partial(
    pl.kernel,
    mesh=_sc_mesh,
    out_type=jax.ShapeDtypeStruct((_B, 2 * _H), jnp.float32),
    scratch_types=[
        pltpu.VMEM((_BPW,), jnp.int32),
        pltpu.VMEM((_BPW,), jnp.int32),
        pltpu.VMEM((_BPW, 2 * _H), jnp.float32),
        pltpu.SemaphoreType.DMA,
    ],
)
def _sc_gather(table_hbm, idx_hbm, out_hbm, idx_v, pidx_v, rows_v, sem):
    wid = lax.axis_index("s") * _NC + lax.axis_index("c")
    base = wid * _BPW
    pltpu.sync_copy(idx_hbm.at[pl.ds(base, _BPW)], idx_v)

    @pl.loop(0, _BPW, step=_LANES)
    def _(i):
        slc = pl.ds(i, _LANES)
        pidx_v.at[slc][...] = lax.shift_right_logical(idx_v.at[slc][...], 1)

    pltpu.async_copy(table_hbm.at[pidx_v], rows_v, sem).wait()
    pltpu.sync_copy(rows_v, out_hbm.at[pl.ds(base, _BPW)])


def _select_half(g, xi):
    odd = (xi & 1) == 1
    return jnp.where(odd, g[:, _H:], g[:, :_H])


def _make_mm_body(ng):
    def _mm_body(prev_ref, g_ref, xi_ref, w_ref, o_hbm, h_ref, obuf, sems):
        del prev_ref
        i = pl.program_id(0)
        slot = lax.rem(i, _NBUF)

        @pl.when(i == 0)
        def _():
            h = _select_half(g_ref[...], xi_ref[...])
            h_ref[...] = h.astype(jnp.bfloat16)

        @pl.when(i >= _NBUF)
        def _():
            pltpu.make_async_copy(
                obuf.at[slot],
                o_hbm.at[:, pl.ds((i - _NBUF) * _BN, _BN)],
                sems.at[slot],
            ).wait()

        obuf[slot] = lax.dot_general(
            h_ref[...],
            w_ref[...].astype(jnp.bfloat16),
            dimension_numbers=(((1,), (1,)), ((), ())),
            preferred_element_type=jnp.float32,
        )
        pltpu.make_async_copy(
            obuf.at[slot],
            o_hbm.at[:, pl.ds(i * _BN, _BN)],
            sems.at[slot],
        ).start()

        @pl.when(i == ng - 1)
        def _():
            for k in range(_NBUF):
                j = ng - _NBUF + k
                pltpu.make_async_copy(
                    obuf.at[j % _NBUF],
                    o_hbm.at[:, pl.ds(j * _BN, _BN)],
                    sems.at[j % _NBUF],
                ).wait()

    return _mm_body


def _tail_body(g_ref, xi_ref, w_ref, o_ref):
    h = _select_half(g_ref[...], xi_ref[...])
    o_ref[...] = lax.dot_general(
        h.astype(jnp.bfloat16),
        w_ref[...].astype(jnp.bfloat16),
        dimension_numbers=(((1,), (1,)), ((), ())),
        preferred_element_type=jnp.float32,
    )


def kernel(x, emb, W):
    xi = x.astype(jnp.int32)
    xi2 = xi.reshape(_B, 1)
    table = emb.reshape(emb.shape[0] // 2, 2 * _H)
    g = _sc_gather(table, xi)
    V = W.shape[0]
    ng = (V // _BN)
    v_tail = V - ng * _BN

    bt = 256
    tail_block = ng * _BN // bt
    tail_out = pl.pallas_call(
        _tail_body,
        grid=(1,),
        in_specs=[
            pl.BlockSpec((_B, 2 * _H), lambda i: (0, 0)),
            pl.BlockSpec((_B, 1), lambda i: (0, 0)),
            pl.BlockSpec((bt, _H), lambda i: (tail_block, 0)),
        ],
        out_specs=pl.BlockSpec((_B, bt), lambda i: (0, tail_block)),
        out_shape=jax.ShapeDtypeStruct((_B, V), jnp.float32),
    )(g, xi2, W)

    logits = pl.pallas_call(
        _make_mm_body(ng),
        grid=(ng,),
        in_specs=[
            pl.BlockSpec(memory_space=pl.ANY),
            pl.BlockSpec((_B, 2 * _H), lambda i: (0, 0)),
            pl.BlockSpec((_B, 1), lambda i: (0, 0)),
            pl.BlockSpec((_BN, _H), lambda i: (i, 0)),
        ],
        out_specs=pl.BlockSpec(memory_space=pl.ANY),
        out_shape=jax.ShapeDtypeStruct((_B, V), jnp.float32),
        scratch_shapes=[
            pltpu.VMEM((_B, _H), jnp.bfloat16),
            pltpu.VMEM((_NBUF, _B, _BN), jnp.float32),
            pltpu.SemaphoreType.DMA((_NBUF,)),
        ],
        compiler_params=pltpu.CompilerParams(
            dimension_semantics=("arbitrary",),
        ),
        input_output_aliases={0: 0},
    )(tail_out, g, xi2, W)
    return logits


# speedup vs baseline: 2.6393x; 2.6383x over previous
import functools

import jax
import jax.numpy as jnp
from jax import lax
from jax.experimental import pallas as pl
from jax.experimental.pallas import tpu as pltpu
from jax.experimental.pallas import tpu_sc as plsc

_B = 1024
_H = 64
_NSC = 2
_HALF = _B // _NSC

_BN = 512
_NBUF = 8

_scalar_mesh = plsc.ScalarSubcoreMesh(axis_name="core", num_cores=_NSC)


@functools.partial(
    pl.kernel,
    mesh=_scalar_mesh,
    out_type=jax.ShapeDtypeStruct((_B, _H), jnp.float32),
    scratch_types=[
        pltpu.SMEM((_HALF,), jnp.int32),
        pltpu.SemaphoreType.DMA,
        pltpu.SemaphoreType.DMA,
    ],
)
def _sc_gather(table_hbm, idx_hbm, out_hbm, idx_s, isem, gsem):
    cid = lax.axis_index("core")
    base = cid * _HALF
    pltpu.async_copy(idx_hbm.at[pl.ds(base, _HALF)], idx_s, isem).wait()

    @pl.loop(0, _HALF)
    def _(i):
        pltpu.make_async_copy(
            table_hbm.at[idx_s[i]], out_hbm.at[base + i], gsem
        ).start()

    @pl.loop(0, _HALF)
    def _(i):
        pltpu.make_async_copy(
            table_hbm.at[0], out_hbm.at[base], gsem
        ).wait()


def _make_mm_body(ng, v_tail):
    def _mm_body(h_ref, wt_ref, o_hbm, ht_ref, obuf, sems):
        i = pl.program_id(0)
        slot = lax.rem(i, _NBUF)

        @pl.when(i == 0)
        def _():
            ht_ref[...] = jnp.transpose(h_ref[...]).astype(jnp.bfloat16)

        @pl.when(i >= _NBUF)
        def _():
            pltpu.make_async_copy(
                obuf.at[slot],
                o_hbm.at[pl.ds((i - _NBUF) * _BN, _BN)],
                sems.at[slot],
            ).wait()

        obuf[slot] = lax.dot_general(
            wt_ref[...].astype(jnp.bfloat16),
            ht_ref[...],
            dimension_numbers=(((0,), (0,)), ((), ())),
            preferred_element_type=jnp.float32,
        )

        @pl.when(i < ng - 1)
        def _():
            pltpu.make_async_copy(
                obuf.at[slot],
                o_hbm.at[pl.ds(i * _BN, _BN)],
                sems.at[slot],
            ).start()

        @pl.when(i == ng - 1)
        def _():
            pltpu.make_async_copy(
                obuf.at[slot, pl.ds(0, v_tail)],
                o_hbm.at[pl.ds(i * _BN, v_tail)],
                sems.at[slot],
            ).start()
            for k in range(_NBUF - 1):
                j = ng - _NBUF + k
                pltpu.make_async_copy(
                    obuf.at[j % _NBUF],
                    o_hbm.at[pl.ds(j * _BN, _BN)],
                    sems.at[j % _NBUF],
                ).wait()
            pltpu.make_async_copy(
                obuf.at[slot, pl.ds(0, v_tail)],
                o_hbm.at[pl.ds(i * _BN, v_tail)],
                sems.at[slot],
            ).wait()

    return _mm_body


def kernel(x, emb, W):
    xi = x.astype(jnp.int32)
    h = _sc_gather(emb, xi)
    V = W.shape[0]
    wt = W.T
    ng = pl.cdiv(V, _BN)
    v_tail = V - (ng - 1) * _BN

    lt = pl.pallas_call(
        _make_mm_body(ng, v_tail),
        grid=(ng,),
        in_specs=[
            pl.BlockSpec((_B, _H), lambda i: (0, 0)),
            pl.BlockSpec((_H, _BN), lambda i: (0, i)),
        ],
        out_specs=pl.BlockSpec(memory_space=pl.ANY),
        out_shape=jax.ShapeDtypeStruct((V, _B), jnp.float32),
        scratch_shapes=[
            pltpu.VMEM((_H, _B), jnp.bfloat16),
            pltpu.VMEM((_NBUF, _BN, _B), jnp.float32),
            pltpu.SemaphoreType.DMA((_NBUF,)),
        ],
        compiler_params=pltpu.CompilerParams(
            dimension_semantics=("arbitrary",),
        ),
    )(h, wt)
    return lt.T
